# baseline (speedup 1.0000x reference)
import functools

import jax
import jax.numpy as jnp
from jax import lax
from jax.experimental import pallas as pl
from jax.experimental.pallas import tpu as pltpu
from jax.experimental.pallas import tpu_sc as plsc

_N = 10000
_NP = 10240
_E = 320000
_D = 128

_NC = 2
_NS = 16
_CHUNK = 120
_EDGES_PER_TILE = _E // (_NC * _NS)
_FULL = _EDGES_PER_TILE // _CHUNK
_TAIL = _EDGES_PER_TILE - _FULL * _CHUNK
_ROWS_PER_TILE = _NP // _NS
_ZR = 80


def _sc_body(x_hbm, edge_hbm, p_out, deg_out,
             col_idx_v, row_idx_v, rows_v, ones_v, tcol_v, trow_v,
             acc_sh, deg_sh, sem_i, sem_g, sem_s, sem_z):
    c = lax.axis_index("c")
    s = lax.axis_index("s")
    r0 = s * _ROWS_PER_TILE

    tile_base = (c * _NS + s) * _EDGES_PER_TILE
    for i in range(_CHUNK // 16 + 1):
        ones_v[pl.ds(i * 16, 16)] = jnp.ones((16,), jnp.float32)

    z16 = jnp.zeros((16,), jnp.float32)

    def zrow(i, carry):
        for o in range(_D // 16):
            rows_v[0, i, pl.ds(o * 16, 16)] = z16
        return carry

    lax.fori_loop(0, _ZR, zrow, 0)
    for t in range(_ROWS_PER_TILE // _ZR):
        pltpu.async_copy(rows_v.at[0, pl.ds(0, _ZR)],
                         acc_sh.at[pl.ds(r0 + t * _ZR, _ZR)], sem_z)
    for t in range(_ROWS_PER_TILE // _D):
        pltpu.async_copy(rows_v.at[0, 0], deg_sh.at[pl.ds(r0 + t * _D, _D)], sem_z)

    def zero_wait():
        for t in range(_ROWS_PER_TILE // _ZR):
            pltpu.make_async_copy(rows_v.at[0, pl.ds(0, _ZR)],
                                  acc_sh.at[pl.ds(0, _ZR)], sem_z).wait()
        for t in range(_ROWS_PER_TILE // _D):
            pltpu.make_async_copy(rows_v.at[0, 0], deg_sh.at[pl.ds(0, _D)], sem_z).wait()

    def idx_start(k, sc_, sr):
        base = tile_base + k * _CHUNK
        pltpu.async_copy(edge_hbm.at[pl.ds(_E + base, _CHUNK)], col_idx_v.at[sc_], sem_i)
        pltpu.async_copy(edge_hbm.at[pl.ds(base, _CHUNK)], row_idx_v.at[sr], sem_i)

    def idx_wait():
        pltpu.make_async_copy(edge_hbm.at[pl.ds(0, _CHUNK)], col_idx_v.at[0], sem_i).wait()
        pltpu.make_async_copy(edge_hbm.at[pl.ds(0, _CHUNK)], row_idx_v.at[0], sem_i).wait()

    def gather_start(b, si):
        pltpu.async_copy(x_hbm.at[col_idx_v.at[si]], rows_v.at[b], sem_g)

    def gather_wait(b):
        pltpu.make_async_copy(x_hbm.at[pl.ds(0, _CHUNK)], rows_v.at[b], sem_g).wait()

    def scatter_start(b, si):
        pltpu.async_copy(rows_v.at[b], acc_sh.at[row_idx_v.at[si]], sem_s, add=True)
        pltpu.async_copy(ones_v.at[pl.ds(0, _CHUNK)], deg_sh.at[row_idx_v.at[si]],
                         sem_s, add=True)

    def scatter_wait():
        pltpu.make_async_copy(rows_v.at[0], acc_sh.at[pl.ds(0, _CHUNK)], sem_s).wait()
        pltpu.make_async_copy(ones_v.at[pl.ds(0, _CHUNK)],
                              deg_sh.at[pl.ds(0, _CHUNK)], sem_s).wait()

    def step(k):
        if k >= 2:
            scatter_wait()
        if k + 2 <= _FULL - 1:
            idx_start(k + 2, (k + 2) % 3, (k + 2) % 6)
        if k + 1 <= _FULL - 1:
            idx_wait()
            gather_start((k + 1) % 3, (k + 1) % 3)
        gather_wait(k % 3)
        scatter_start(k % 3, k % 6)

    idx_start(0, 0, 0)
    idx_start(1, 1, 1)
    idx_wait()
    zero_wait()
    plsc.subcore_barrier()
    gather_start(0, 0)
    step(0)
    step(1)

    def body6(j, carry):
        k0 = 6 * j + 2
        for o in range(6):
            k = k0 + o
            scatter_wait()
            idx_start(k + 2, (2 + o + 2) % 3, (2 + o + 2) % 6)
            idx_wait()
            gather_start((2 + o + 1) % 3, (2 + o + 1) % 3)
            gather_wait((2 + o) % 3)
            scatter_start((2 + o) % 3, (2 + o) % 6)
        return carry

    lax.fori_loop(0, (_FULL - 5) // 6, body6, 0)

    for k in range(_FULL - 3, _FULL):
        step(k)
    scatter_wait()
    scatter_wait()

    tbase = tile_base + _FULL * _CHUNK
    pltpu.async_copy(edge_hbm.at[pl.ds(_E + tbase, _TAIL)], tcol_v, sem_i)
    pltpu.async_copy(edge_hbm.at[pl.ds(tbase, _TAIL)], trow_v, sem_i)
    pltpu.make_async_copy(edge_hbm.at[pl.ds(0, _TAIL)], tcol_v, sem_i).wait()
    pltpu.make_async_copy(edge_hbm.at[pl.ds(0, _TAIL)], trow_v, sem_i).wait()
    pltpu.async_copy(x_hbm.at[tcol_v], rows_v.at[0, pl.ds(0, _TAIL)], sem_g).wait()
    pltpu.async_copy(rows_v.at[0, pl.ds(0, _TAIL)],
                     acc_sh.at[trow_v], sem_s, add=True)
    pltpu.async_copy(ones_v.at[pl.ds(0, _TAIL)], deg_sh.at[trow_v], sem_s, add=True)
    pltpu.make_async_copy(rows_v.at[0, pl.ds(0, _TAIL)],
                          acc_sh.at[pl.ds(0, _TAIL)], sem_s).wait()
    pltpu.make_async_copy(ones_v.at[pl.ds(0, _TAIL)],
                          deg_sh.at[pl.ds(0, _TAIL)], sem_s).wait()

    plsc.subcore_barrier()

    pltpu.sync_copy(acc_sh.at[pl.ds(r0, _ROWS_PER_TILE)],
                    p_out.at[c, pl.ds(r0, _ROWS_PER_TILE)])
    pltpu.sync_copy(deg_sh.at[pl.ds(r0, _ROWS_PER_TILE)],
                    deg_out.at[c, pl.ds(r0, _ROWS_PER_TILE)])


_sc_aggregate = functools.partial(
    pl.kernel,
    out_type=(
        jax.ShapeDtypeStruct((_NC, _NP, _D), jnp.float32),
        jax.ShapeDtypeStruct((_NC, _NP), jnp.float32),
    ),
    mesh=plsc.VectorSubcoreMesh(core_axis_name="c", subcore_axis_name="s"),
    compiler_params=pltpu.CompilerParams(use_tc_tiling_on_sc=False),
    scratch_types=[
        pltpu.VMEM((3, _CHUNK), jnp.int32),
        pltpu.VMEM((6, _CHUNK), jnp.int32),
        pltpu.VMEM((3, _CHUNK, _D), jnp.float32),
        pltpu.VMEM((_CHUNK + 16,), jnp.float32),
        pltpu.VMEM((_TAIL,), jnp.int32),
        pltpu.VMEM((_TAIL,), jnp.int32),
        pltpu.VMEM_SHARED((_NP, _D), jnp.float32),
        pltpu.VMEM_SHARED((_NP,), jnp.float32),
        pltpu.SemaphoreType.DMA,
        pltpu.SemaphoreType.DMA,
        pltpu.SemaphoreType.DMA,
        pltpu.SemaphoreType.DMA,
    ],
)(_sc_body)


def _tc_body(p_ref, d_ref, w_ref, o_ref):
    agg = p_ref[0] + p_ref[1]
    deg = d_ref[0] + d_ref[1]
    deg = deg + jnp.where(deg == 0.0, 1.0, 0.0)
    h = jnp.dot(agg, w_ref[...], preferred_element_type=jnp.float32)
    o_ref[...] = h / deg


_ROWS_BLK = 1000


def _tc_finish(p, deg, w):
    grid = _N // _ROWS_BLK
    return pl.pallas_call(
        _tc_body,
        grid=(grid,),
        in_specs=[
            pl.BlockSpec((_NC, _ROWS_BLK, _D), lambda i: (0, i, 0)),
            pl.BlockSpec((_NC, _ROWS_BLK, 1), lambda i: (0, i, 0)),
            pl.BlockSpec((_D, _D), lambda i: (0, 0)),
        ],
        out_specs=pl.BlockSpec((_ROWS_BLK, _D), lambda i: (i, 0)),
        out_shape=jax.ShapeDtypeStruct((_N, _D), jnp.float32),
    )(p, deg, w)


def kernel(input, edge_index, W):
    edge_flat = edge_index.reshape(2 * _E)
    p, deg = _sc_aggregate(input, edge_flat)
    return _tc_finish(p, deg.reshape(_NC, _NP, 1), W)


# speedup vs baseline: 1.0339x; 1.0339x over previous
import functools

import jax
import jax.numpy as jnp
from jax import lax
from jax.experimental import pallas as pl
from jax.experimental.pallas import tpu as pltpu
from jax.experimental.pallas import tpu_sc as plsc

_N = 10000
_NP = 10240
_E = 320000
_D = 128

_NC = 2
_NS = 16
_CHUNK = 120
_EDGES_PER_TILE = _E // (_NC * _NS)
_FULL = _EDGES_PER_TILE // _CHUNK
_TAIL = _EDGES_PER_TILE - _FULL * _CHUNK
_ROWS_PER_TILE = _NP // _NS
_ZR = 80


def _sc_body(x_hbm, edge_hbm, p_out, deg_out,
             col_idx_v, row_idx_v, rows_v, ones_v, tcol_v, trow_v,
             acc_sh, deg_sh, sem_i, sem_g, sem_s, sem_z):
    c = lax.axis_index("c")
    s = lax.axis_index("s")
    r0 = s * _ROWS_PER_TILE

    tile_base = (c * _NS + s) * _EDGES_PER_TILE
    for i in range(_CHUNK // 16 + 1):
        ones_v[pl.ds(i * 16, 16)] = jnp.ones((16,), jnp.float32)

    z16 = jnp.zeros((16,), jnp.float32)

    def zrow(i, carry):
        for o in range(_D // 16):
            rows_v[0, i, pl.ds(o * 16, 16)] = z16
        return carry

    lax.fori_loop(0, _ZR, zrow, 0)
    for t in range(_ROWS_PER_TILE // _ZR):
        pltpu.async_copy(rows_v.at[0, pl.ds(0, _ZR)],
                         acc_sh.at[pl.ds(r0 + t * _ZR, _ZR)], sem_z)
    for t in range(_ROWS_PER_TILE // _D):
        pltpu.async_copy(rows_v.at[0, 0], deg_sh.at[pl.ds(r0 + t * _D, _D)], sem_z)

    def zero_wait():
        for t in range(_ROWS_PER_TILE // _ZR):
            pltpu.make_async_copy(rows_v.at[0, pl.ds(0, _ZR)],
                                  acc_sh.at[pl.ds(0, _ZR)], sem_z).wait()
        for t in range(_ROWS_PER_TILE // _D):
            pltpu.make_async_copy(rows_v.at[0, 0], deg_sh.at[pl.ds(0, _D)], sem_z).wait()

    def idx_start(k, sc_, sr):
        base = tile_base + k * _CHUNK
        pltpu.async_copy(edge_hbm.at[pl.ds(_E + base, _CHUNK)], col_idx_v.at[sc_], sem_i)
        pltpu.async_copy(edge_hbm.at[pl.ds(base, _CHUNK)], row_idx_v.at[sr], sem_i)

    def idx_wait():
        pltpu.make_async_copy(edge_hbm.at[pl.ds(0, _CHUNK)], col_idx_v.at[0], sem_i).wait()
        pltpu.make_async_copy(edge_hbm.at[pl.ds(0, _CHUNK)], row_idx_v.at[0], sem_i).wait()

    def gather_start(b, si):
        pltpu.async_copy(x_hbm.at[col_idx_v.at[si]], rows_v.at[b], sem_g)

    def gather_wait(b):
        pltpu.make_async_copy(x_hbm.at[pl.ds(0, _CHUNK)], rows_v.at[b], sem_g).wait()

    def scatter_start(b, si):
        pltpu.async_copy(rows_v.at[b], acc_sh.at[row_idx_v.at[si]], sem_s, add=True)
        pltpu.async_copy(ones_v.at[pl.ds(0, _CHUNK)], deg_sh.at[row_idx_v.at[si]],
                         sem_s, add=True)

    def scatter_wait():
        pltpu.make_async_copy(rows_v.at[0], acc_sh.at[pl.ds(0, _CHUNK)], sem_s).wait()
        pltpu.make_async_copy(ones_v.at[pl.ds(0, _CHUNK)],
                              deg_sh.at[pl.ds(0, _CHUNK)], sem_s).wait()

    def step(k):
        if k + 2 <= _FULL - 1:
            idx_start(k + 2, (k + 2) % 3, (k + 2) % 6)
        if k >= 2:
            scatter_wait()
        if k + 1 <= _FULL - 1:
            idx_wait()
            gather_start((k + 1) % 3, (k + 1) % 3)
        gather_wait(k % 3)
        scatter_start(k % 3, k % 6)

    idx_start(0, 0, 0)
    idx_start(1, 1, 1)
    idx_wait()
    zero_wait()
    plsc.subcore_barrier()
    gather_start(0, 0)
    step(0)
    step(1)

    def body6(j, carry):
        k0 = 6 * j + 2
        for o in range(6):
            k = k0 + o
            idx_start(k + 2, (2 + o + 2) % 3, (2 + o + 2) % 6)
            scatter_wait()
            idx_wait()
            gather_start((2 + o + 1) % 3, (2 + o + 1) % 3)
            gather_wait((2 + o) % 3)
            scatter_start((2 + o) % 3, (2 + o) % 6)
        return carry

    lax.fori_loop(0, (_FULL - 5) // 6, body6, 0)

    for k in range(_FULL - 3, _FULL):
        step(k)
    scatter_wait()
    scatter_wait()

    tbase = tile_base + _FULL * _CHUNK
    pltpu.async_copy(edge_hbm.at[pl.ds(_E + tbase, _TAIL)], tcol_v, sem_i)
    pltpu.async_copy(edge_hbm.at[pl.ds(tbase, _TAIL)], trow_v, sem_i)
    pltpu.make_async_copy(edge_hbm.at[pl.ds(0, _TAIL)], tcol_v, sem_i).wait()
    pltpu.make_async_copy(edge_hbm.at[pl.ds(0, _TAIL)], trow_v, sem_i).wait()
    pltpu.async_copy(x_hbm.at[tcol_v], rows_v.at[0, pl.ds(0, _TAIL)], sem_g).wait()
    pltpu.async_copy(rows_v.at[0, pl.ds(0, _TAIL)],
                     acc_sh.at[trow_v], sem_s, add=True)
    pltpu.async_copy(ones_v.at[pl.ds(0, _TAIL)], deg_sh.at[trow_v], sem_s, add=True)
    pltpu.make_async_copy(rows_v.at[0, pl.ds(0, _TAIL)],
                          acc_sh.at[pl.ds(0, _TAIL)], sem_s).wait()
    pltpu.make_async_copy(ones_v.at[pl.ds(0, _TAIL)],
                          deg_sh.at[pl.ds(0, _TAIL)], sem_s).wait()

    plsc.subcore_barrier()

    pltpu.sync_copy(acc_sh.at[pl.ds(r0, _ROWS_PER_TILE)],
                    p_out.at[c, pl.ds(r0, _ROWS_PER_TILE)])
    pltpu.sync_copy(deg_sh.at[pl.ds(r0, _ROWS_PER_TILE)],
                    deg_out.at[c, pl.ds(r0, _ROWS_PER_TILE)])


_sc_aggregate = functools.partial(
    pl.kernel,
    out_type=(
        jax.ShapeDtypeStruct((_NC, _NP, _D), jnp.float32),
        jax.ShapeDtypeStruct((_NC, _NP), jnp.float32),
    ),
    mesh=plsc.VectorSubcoreMesh(core_axis_name="c", subcore_axis_name="s"),
    scratch_types=[
        pltpu.VMEM((3, _CHUNK), jnp.int32),
        pltpu.VMEM((6, _CHUNK), jnp.int32),
        pltpu.VMEM((3, _CHUNK, _D), jnp.float32),
        pltpu.VMEM((_CHUNK + 16,), jnp.float32),
        pltpu.VMEM((_TAIL,), jnp.int32),
        pltpu.VMEM((_TAIL,), jnp.int32),
        pltpu.VMEM_SHARED((_NP, _D), jnp.float32),
        pltpu.VMEM_SHARED((_NP,), jnp.float32),
        pltpu.SemaphoreType.DMA,
        pltpu.SemaphoreType.DMA,
        pltpu.SemaphoreType.DMA,
        pltpu.SemaphoreType.DMA,
    ],
)(_sc_body)


def _tc_body(p_ref, d_ref, w_ref, o_ref):
    agg = p_ref[0] + p_ref[1]
    deg = d_ref[0] + d_ref[1]
    deg = deg + jnp.where(deg == 0.0, 1.0, 0.0)
    h = jnp.dot(agg, w_ref[...], preferred_element_type=jnp.float32)
    o_ref[...] = h / deg


_ROWS_BLK = 1000


def _tc_finish(p, deg, w):
    grid = _N // _ROWS_BLK
    return pl.pallas_call(
        _tc_body,
        grid=(grid,),
        in_specs=[
            pl.BlockSpec((_NC, _ROWS_BLK, _D), lambda i: (0, i, 0)),
            pl.BlockSpec((_NC, _ROWS_BLK, 1), lambda i: (0, i, 0)),
            pl.BlockSpec((_D, _D), lambda i: (0, 0)),
        ],
        out_specs=pl.BlockSpec((_ROWS_BLK, _D), lambda i: (i, 0)),
        out_shape=jax.ShapeDtypeStruct((_N, _D), jnp.float32),
    )(p, deg, w)


def kernel(input, edge_index, W):
    edge_flat = edge_index.reshape(2 * _E)
    p, deg = _sc_aggregate(input, edge_flat)
    return _tc_finish(p, deg.reshape(_NC, _NP, 1), W)


# speedup vs baseline: 1.0446x; 1.0104x over previous
import functools

import jax
import jax.numpy as jnp
from jax import lax
from jax.experimental import pallas as pl
from jax.experimental.pallas import tpu as pltpu
from jax.experimental.pallas import tpu_sc as plsc

_N = 10000
_NP = 10240
_E = 320000
_D = 128

_NC = 2
_NS = 16
_CHUNK = 120
_EDGES_PER_TILE = _E // (_NC * _NS)
_FULL = _EDGES_PER_TILE // _CHUNK
_TAIL = _EDGES_PER_TILE - _FULL * _CHUNK
_ROWS_PER_TILE = _NP // _NS
_ZR = 80


def _sc_body(x_hbm, edge_hbm, p_out, deg_out,
             col_idx_v, row_idx_v, rows_v, ones_v, tcol_v, trow_v,
             acc_sh, deg_sh, sem_i, sem_g, sem_s, sem_z):
    c = lax.axis_index("c")
    s = lax.axis_index("s")
    r0 = s * _ROWS_PER_TILE

    tile_base = (c * _NS + s) * _EDGES_PER_TILE
    for i in range(_CHUNK // 16 + 1):
        ones_v[pl.ds(i * 16, 16)] = jnp.ones((16,), jnp.float32)

    z16 = jnp.zeros((16,), jnp.float32)

    def zrow(i, carry):
        for o in range(_D // 16):
            rows_v[0, i, pl.ds(o * 16, 16)] = z16
        return carry

    lax.fori_loop(0, _ZR, zrow, 0)
    for t in range(_ROWS_PER_TILE // _ZR):
        pltpu.async_copy(rows_v.at[0, pl.ds(0, _ZR)],
                         acc_sh.at[pl.ds(r0 + t * _ZR, _ZR)], sem_z)
    for t in range(_ROWS_PER_TILE // _D):
        pltpu.async_copy(rows_v.at[0, 0], deg_sh.at[pl.ds(r0 + t * _D, _D)], sem_z)

    def zero_wait():
        for t in range(_ROWS_PER_TILE // _ZR):
            pltpu.make_async_copy(rows_v.at[0, pl.ds(0, _ZR)],
                                  acc_sh.at[pl.ds(0, _ZR)], sem_z).wait()
        for t in range(_ROWS_PER_TILE // _D):
            pltpu.make_async_copy(rows_v.at[0, 0], deg_sh.at[pl.ds(0, _D)], sem_z).wait()

    def idx_start(k, sc_, sr):
        base = tile_base + k * _CHUNK
        pltpu.async_copy(edge_hbm.at[pl.ds(_E + base, _CHUNK)], col_idx_v.at[sc_], sem_i)
        pltpu.async_copy(edge_hbm.at[pl.ds(base, _CHUNK)], row_idx_v.at[sr], sem_i)

    def idx_wait():
        pltpu.make_async_copy(edge_hbm.at[pl.ds(0, _CHUNK)], col_idx_v.at[0], sem_i).wait()
        pltpu.make_async_copy(edge_hbm.at[pl.ds(0, _CHUNK)], row_idx_v.at[0], sem_i).wait()

    def gather_start(b, si):
        pltpu.async_copy(x_hbm.at[col_idx_v.at[si]], rows_v.at[b], sem_g, priority=1)

    def gather_wait(b):
        pltpu.make_async_copy(x_hbm.at[pl.ds(0, _CHUNK)], rows_v.at[b], sem_g).wait()

    def scatter_start(b, si):
        pltpu.async_copy(rows_v.at[b], acc_sh.at[row_idx_v.at[si]], sem_s, add=True)
        pltpu.async_copy(ones_v.at[pl.ds(0, _CHUNK)], deg_sh.at[row_idx_v.at[si]],
                         sem_s, add=True)

    def scatter_wait():
        pltpu.make_async_copy(rows_v.at[0], acc_sh.at[pl.ds(0, _CHUNK)], sem_s).wait()
        pltpu.make_async_copy(ones_v.at[pl.ds(0, _CHUNK)],
                              deg_sh.at[pl.ds(0, _CHUNK)], sem_s).wait()

    def step(k):
        if k >= 2:
            scatter_wait()
        if k + 2 <= _FULL - 1:
            idx_start(k + 2, (k + 2) % 3, (k + 2) % 6)
        if k + 1 <= _FULL - 1:
            idx_wait()
            gather_start((k + 1) % 3, (k + 1) % 3)
        gather_wait(k % 3)
        scatter_start(k % 3, k % 6)

    idx_start(0, 0, 0)
    idx_start(1, 1, 1)
    idx_wait()
    zero_wait()
    plsc.subcore_barrier()
    gather_start(0, 0)
    step(0)
    step(1)

    def body6(j, carry):
        k0 = 6 * j + 2
        for o in range(6):
            k = k0 + o
            scatter_wait()
            idx_start(k + 2, (2 + o + 2) % 3, (2 + o + 2) % 6)
            idx_wait()
            gather_start((2 + o + 1) % 3, (2 + o + 1) % 3)
            gather_wait((2 + o) % 3)
            scatter_start((2 + o) % 3, (2 + o) % 6)
        return carry

    lax.fori_loop(0, (_FULL - 5) // 6, body6, 0)

    for k in range(_FULL - 3, _FULL):
        step(k)
    scatter_wait()
    scatter_wait()

    tbase = tile_base + _FULL * _CHUNK
    pltpu.async_copy(edge_hbm.at[pl.ds(_E + tbase, _TAIL)], tcol_v, sem_i)
    pltpu.async_copy(edge_hbm.at[pl.ds(tbase, _TAIL)], trow_v, sem_i)
    pltpu.make_async_copy(edge_hbm.at[pl.ds(0, _TAIL)], tcol_v, sem_i).wait()
    pltpu.make_async_copy(edge_hbm.at[pl.ds(0, _TAIL)], trow_v, sem_i).wait()
    pltpu.async_copy(x_hbm.at[tcol_v], rows_v.at[0, pl.ds(0, _TAIL)], sem_g).wait()
    pltpu.async_copy(rows_v.at[0, pl.ds(0, _TAIL)],
                     acc_sh.at[trow_v], sem_s, add=True)
    pltpu.async_copy(ones_v.at[pl.ds(0, _TAIL)], deg_sh.at[trow_v], sem_s, add=True)
    pltpu.make_async_copy(rows_v.at[0, pl.ds(0, _TAIL)],
                          acc_sh.at[pl.ds(0, _TAIL)], sem_s).wait()
    pltpu.make_async_copy(ones_v.at[pl.ds(0, _TAIL)],
                          deg_sh.at[pl.ds(0, _TAIL)], sem_s).wait()

    plsc.subcore_barrier()

    pltpu.sync_copy(acc_sh.at[pl.ds(r0, _ROWS_PER_TILE)],
                    p_out.at[c, pl.ds(r0, _ROWS_PER_TILE)])
    pltpu.sync_copy(deg_sh.at[pl.ds(r0, _ROWS_PER_TILE)],
                    deg_out.at[c, pl.ds(r0, _ROWS_PER_TILE)])


_sc_aggregate = functools.partial(
    pl.kernel,
    out_type=(
        jax.ShapeDtypeStruct((_NC, _NP, _D), jnp.float32),
        jax.ShapeDtypeStruct((_NC, _NP), jnp.float32),
    ),
    mesh=plsc.VectorSubcoreMesh(core_axis_name="c", subcore_axis_name="s"),
    scratch_types=[
        pltpu.VMEM((3, _CHUNK), jnp.int32),
        pltpu.VMEM((6, _CHUNK), jnp.int32),
        pltpu.VMEM((3, _CHUNK, _D), jnp.float32),
        pltpu.VMEM((_CHUNK + 16,), jnp.float32),
        pltpu.VMEM((_TAIL,), jnp.int32),
        pltpu.VMEM((_TAIL,), jnp.int32),
        pltpu.VMEM_SHARED((_NP, _D), jnp.float32),
        pltpu.VMEM_SHARED((_NP,), jnp.float32),
        pltpu.SemaphoreType.DMA,
        pltpu.SemaphoreType.DMA,
        pltpu.SemaphoreType.DMA,
        pltpu.SemaphoreType.DMA,
    ],
)(_sc_body)


def _tc_body(p_ref, d_ref, w_ref, o_ref):
    agg = p_ref[0] + p_ref[1]
    deg = d_ref[0] + d_ref[1]
    deg = deg + jnp.where(deg == 0.0, 1.0, 0.0)
    h = jnp.dot(agg, w_ref[...], preferred_element_type=jnp.float32)
    o_ref[...] = h / deg


_ROWS_BLK = 1000


def _tc_finish(p, deg, w):
    grid = _N // _ROWS_BLK
    return pl.pallas_call(
        _tc_body,
        grid=(grid,),
        in_specs=[
            pl.BlockSpec((_NC, _ROWS_BLK, _D), lambda i: (0, i, 0)),
            pl.BlockSpec((_NC, _ROWS_BLK, 1), lambda i: (0, i, 0)),
            pl.BlockSpec((_D, _D), lambda i: (0, 0)),
        ],
        out_specs=pl.BlockSpec((_ROWS_BLK, _D), lambda i: (i, 0)),
        out_shape=jax.ShapeDtypeStruct((_N, _D), jnp.float32),
    )(p, deg, w)


def kernel(input, edge_index, W):
    edge_flat = edge_index.reshape(2 * _E)
    p, deg = _sc_aggregate(input, edge_flat)
    return _tc_finish(p, deg.reshape(_NC, _NP, 1), W)


# speedup vs baseline: 1.0487x; 1.0040x over previous
import functools

import jax
import jax.numpy as jnp
from jax import lax
from jax.experimental import pallas as pl
from jax.experimental.pallas import tpu as pltpu
from jax.experimental.pallas import tpu_sc as plsc

_N = 10000
_NP = 10240
_E = 320000
_D = 128

_NC = 2
_NS = 16
_CHUNK = 120
_EDGES_PER_TILE = _E // (_NC * _NS)
_FULL = _EDGES_PER_TILE // _CHUNK
_TAIL = _EDGES_PER_TILE - _FULL * _CHUNK
_ROWS_PER_TILE = _NP // _NS
_ZR = 80


def _sc_body(x_hbm, edge_hbm, p_out, deg_out,
             col_idx_v, row_idx_v, rows_v, ones_v, tcol_v, trow_v,
             acc_sh, deg_sh, sem_i, sem_g, sem_s, sem_z):
    c = lax.axis_index("c")
    s = lax.axis_index("s")
    r0 = s * _ROWS_PER_TILE

    tile_base = (c * _NS + s) * _EDGES_PER_TILE
    for i in range(_CHUNK // 16 + 1):
        ones_v[pl.ds(i * 16, 16)] = jnp.ones((16,), jnp.float32)

    z16 = jnp.zeros((16,), jnp.float32)

    def zrow(i, carry):
        for o in range(_D // 16):
            rows_v[0, i, pl.ds(o * 16, 16)] = z16
        return carry

    lax.fori_loop(0, _ZR, zrow, 0)
    for t in range(_ROWS_PER_TILE // _ZR):
        pltpu.async_copy(rows_v.at[0, pl.ds(0, _ZR)],
                         acc_sh.at[pl.ds(r0 + t * _ZR, _ZR)], sem_z)
    for t in range(_ROWS_PER_TILE // _D):
        pltpu.async_copy(rows_v.at[0, 0], deg_sh.at[pl.ds(r0 + t * _D, _D)], sem_z)

    def zero_wait():
        for t in range(_ROWS_PER_TILE // _ZR):
            pltpu.make_async_copy(rows_v.at[0, pl.ds(0, _ZR)],
                                  acc_sh.at[pl.ds(0, _ZR)], sem_z).wait()
        for t in range(_ROWS_PER_TILE // _D):
            pltpu.make_async_copy(rows_v.at[0, 0], deg_sh.at[pl.ds(0, _D)], sem_z).wait()

    def idx_start(k, sc_, sr):
        base = tile_base + k * _CHUNK
        pltpu.async_copy(edge_hbm.at[pl.ds(_E + base, _CHUNK)], col_idx_v.at[sc_], sem_i)
        pltpu.async_copy(edge_hbm.at[pl.ds(base, _CHUNK)], row_idx_v.at[sr], sem_i)

    def idx_wait():
        pltpu.make_async_copy(edge_hbm.at[pl.ds(0, _CHUNK)], col_idx_v.at[0], sem_i).wait()
        pltpu.make_async_copy(edge_hbm.at[pl.ds(0, _CHUNK)], row_idx_v.at[0], sem_i).wait()

    def gather_start(b, si):
        pltpu.async_copy(x_hbm.at[col_idx_v.at[si]], rows_v.at[b], sem_g)

    def gather_wait(b):
        pltpu.make_async_copy(x_hbm.at[pl.ds(0, _CHUNK)], rows_v.at[b], sem_g).wait()

    def scatter_start(b, si):
        pltpu.async_copy(rows_v.at[b], acc_sh.at[row_idx_v.at[si]], sem_s, add=True)
        pltpu.async_copy(ones_v.at[pl.ds(0, _CHUNK)], deg_sh.at[row_idx_v.at[si]],
                         sem_s, add=True)

    def scatter_wait():
        pltpu.make_async_copy(rows_v.at[0], acc_sh.at[pl.ds(0, _CHUNK)], sem_s).wait()
        pltpu.make_async_copy(ones_v.at[pl.ds(0, _CHUNK)],
                              deg_sh.at[pl.ds(0, _CHUNK)], sem_s).wait()

    def step(k):
        if k >= 2:
            scatter_wait()
        if k + 2 <= _FULL - 1:
            idx_start(k + 2, (k + 2) % 3, (k + 2) % 6)
        if k + 1 <= _FULL - 1:
            idx_wait()
            gather_start((k + 1) % 3, (k + 1) % 3)
        gather_wait(k % 3)
        scatter_start(k % 3, k % 6)

    idx_start(0, 0, 0)
    idx_start(1, 1, 1)
    idx_wait()
    zero_wait()
    plsc.subcore_barrier()
    gather_start(0, 0)
    step(0)
    step(1)

    def body6(j, carry):
        k0 = 6 * j + 2
        for o in range(6):
            k = k0 + o
            scatter_wait()
            idx_start(k + 2, (2 + o + 2) % 3, (2 + o + 2) % 6)
            idx_wait()
            gather_start((2 + o + 1) % 3, (2 + o + 1) % 3)
            gather_wait((2 + o) % 3)
            scatter_start((2 + o) % 3, (2 + o) % 6)
        return carry

    lax.fori_loop(0, (_FULL - 5) // 6, body6, 0)

    for k in range(_FULL - 3, _FULL):
        step(k)
    scatter_wait()
    scatter_wait()

    tbase = tile_base + _FULL * _CHUNK
    pltpu.async_copy(edge_hbm.at[pl.ds(_E + tbase, _TAIL)], tcol_v, sem_i)
    pltpu.async_copy(edge_hbm.at[pl.ds(tbase, _TAIL)], trow_v, sem_i)
    pltpu.make_async_copy(edge_hbm.at[pl.ds(0, _TAIL)], tcol_v, sem_i).wait()
    pltpu.make_async_copy(edge_hbm.at[pl.ds(0, _TAIL)], trow_v, sem_i).wait()
    pltpu.async_copy(x_hbm.at[tcol_v], rows_v.at[0, pl.ds(0, _TAIL)], sem_g).wait()
    pltpu.async_copy(rows_v.at[0, pl.ds(0, _TAIL)],
                     acc_sh.at[trow_v], sem_s, add=True)
    pltpu.async_copy(ones_v.at[pl.ds(0, _TAIL)], deg_sh.at[trow_v], sem_s, add=True)
    pltpu.make_async_copy(rows_v.at[0, pl.ds(0, _TAIL)],
                          acc_sh.at[pl.ds(0, _TAIL)], sem_s).wait()
    pltpu.make_async_copy(ones_v.at[pl.ds(0, _TAIL)],
                          deg_sh.at[pl.ds(0, _TAIL)], sem_s).wait()

    plsc.subcore_barrier()

    pltpu.sync_copy(acc_sh.at[pl.ds(r0, _ROWS_PER_TILE)],
                    p_out.at[c, pl.ds(r0, _ROWS_PER_TILE)])
    pltpu.sync_copy(deg_sh.at[pl.ds(r0, _ROWS_PER_TILE)],
                    deg_out.at[c, pl.ds(r0, _ROWS_PER_TILE)])


_sc_aggregate = functools.partial(
    pl.kernel,
    out_type=(
        jax.ShapeDtypeStruct((_NC, _NP, _D), jnp.float32),
        jax.ShapeDtypeStruct((_NC, _NP), jnp.float32),
    ),
    mesh=plsc.VectorSubcoreMesh(core_axis_name="c", subcore_axis_name="s"),
    scratch_types=[
        pltpu.VMEM((3, _CHUNK), jnp.int32),
        pltpu.VMEM((6, _CHUNK), jnp.int32),
        pltpu.VMEM((3, _CHUNK, _D), jnp.float32),
        pltpu.VMEM((_CHUNK + 16,), jnp.float32),
        pltpu.VMEM((_TAIL,), jnp.int32),
        pltpu.VMEM((_TAIL,), jnp.int32),
        pltpu.VMEM_SHARED((_NP, _D), jnp.float32),
        pltpu.VMEM_SHARED((_NP,), jnp.float32),
        pltpu.SemaphoreType.DMA,
        pltpu.SemaphoreType.DMA,
        pltpu.SemaphoreType.DMA,
        pltpu.SemaphoreType.DMA,
    ],
)(_sc_body)


def _tc_body(p_ref, d_ref, w_ref, o_ref):
    agg = p_ref[0] + p_ref[1]
    deg = d_ref[0] + d_ref[1]
    deg = deg + jnp.where(deg == 0.0, 1.0, 0.0)
    h = jnp.dot(agg, w_ref[...], preferred_element_type=jnp.float32)
    o_ref[...] = h / deg


_ROWS_BLK = 1000


def _tc_finish(p, deg, w):
    grid = _N // _ROWS_BLK
    return pl.pallas_call(
        _tc_body,
        grid=(grid,),
        in_specs=[
            pl.BlockSpec((_NC, _ROWS_BLK, _D), lambda i: (0, i, 0)),
            pl.BlockSpec((_NC, _ROWS_BLK, 1), lambda i: (0, i, 0)),
            pl.BlockSpec((_D, _D), lambda i: (0, 0)),
        ],
        out_specs=pl.BlockSpec((_ROWS_BLK, _D), lambda i: (i, 0)),
        out_shape=jax.ShapeDtypeStruct((_N, _D), jnp.float32),
    )(p, deg, w)


def kernel(input, edge_index, W):
    edge_flat = edge_index.reshape(2 * _E)
    p, deg = _sc_aggregate(input, edge_flat)
    return _tc_finish(p, deg.reshape(_NC, _NP, 1), W)
